# baseline (speedup 1.0000x reference)
import functools

import jax
import jax.numpy as jnp
from jax import lax
from jax.experimental import pallas as pl
from jax.experimental.pallas import tpu as pltpu
from jax.experimental.pallas import tpu_sc as plsc

N = 10000
E = 320000
H = 16
DE = 16
TSTEPS = 4
PK = 8
EP = E // PK
NP = N // PK

NC = 2
NS = 16
NW = NC * NS
EW = E // NW
CHUNK = 100
NCH = EW // CHUNK
GC1 = 25
NG1 = NCH // GC1
GC2 = 10
NG2 = NCH // GC2
NPT = N // NS

_mesh = plsc.VectorSubcoreMesh(
    core_axis_name="c", subcore_axis_name="s", num_cores=NC, num_subcores=NS
)


@functools.partial(
    pl.kernel,
    out_type=jax.ShapeDtypeStruct((NW, NG1, GC1, CHUNK, H), jnp.float32),
    mesh=_mesh,
    compiler_params=pltpu.CompilerParams(use_tc_tiling_on_sc=False),
    scratch_types=[
        pltpu.VMEM((NCH, CHUNK), jnp.int32),
        pltpu.VMEM((2, GC1, CHUNK, H), jnp.float32),
        pltpu.SemaphoreType.DMA,
        pltpu.SemaphoreType.DMA,
    ],
)
def _sc_gather(h_hbm, src_hbm, hs_hbm, idx_v, rows_v, gsem, wsem):
    c = lax.axis_index("c")
    s = lax.axis_index("s")
    wid = s * NC + c
    pltpu.sync_copy(src_hbm.at[wid], idx_v)

    wdesc = [None, None]
    for g in range(NG1):
        buf = g % 2
        if wdesc[buf] is not None:
            wdesc[buf].wait()
        gds = [
            pltpu.async_copy(
                h_hbm.at[idx_v.at[g * GC1 + k]], rows_v.at[buf, k], gsem
            )
            for k in range(GC1)
        ]
        for d in gds:
            d.wait()
        wdesc[buf] = pltpu.async_copy(rows_v.at[buf], hs_hbm.at[wid, g], wsem)
    for d in wdesc:
        if d is not None:
            d.wait()


NCHG = E // NS // CHUNK
GCG = 10
NGG = NCHG // GCG
GCS = 10
NGS = NCH // GCS


@functools.partial(
    pl.kernel,
    out_type=(
        jax.ShapeDtypeStruct((NC, N, H), jnp.float32),
        jax.ShapeDtypeStruct((NC, NS, NGG, GCG, CHUNK, H), jnp.float32),
    ),
    mesh=_mesh,
    compiler_params=pltpu.CompilerParams(use_tc_tiling_on_sc=False),
    scratch_types=[
        pltpu.VMEM((NCH, CHUNK), jnp.int32),
        pltpu.VMEM((NCHG, CHUNK), jnp.int32),
        pltpu.VMEM((2, GCS, CHUNK, H), jnp.float32),
        pltpu.VMEM((2, GCG, CHUNK, H), jnp.float32),
        pltpu.VMEM((NPT, H), jnp.float32),
        pltpu.VMEM_SHARED((N, H), jnp.float32),
        pltpu.SemaphoreType.DMA,
        pltpu.SemaphoreType.DMA,
        pltpu.SemaphoreType.DMA,
        pltpu.SemaphoreType.DMA,
    ],
)
def _sc_scatter_gather(m_hbm, dst_hbm, srcg_hbm, zeros_hbm, agg_hbm,
                       asrc_hbm, idx_v, idxg_v, mbuf, rows_v, node_v, acc_sh,
                       lsem, ssem, gsem, wsem):
    c = lax.axis_index("c")
    s = lax.axis_index("s")
    wid = s * NC + c
    pltpu.sync_copy(dst_hbm.at[wid], idx_v)
    pltpu.sync_copy(srcg_hbm.at[s], idxg_v)
    pltpu.sync_copy(zeros_hbm, node_v)
    pltpu.sync_copy(node_v, acc_sh.at[pl.ds(s * NPT, NPT)])
    plsc.subcore_barrier()

    ld = [None, None]
    sdescs = [[], []]
    ld[0] = pltpu.async_copy(m_hbm.at[wid, 0], mbuf.at[0], lsem)
    for g in range(NGS):
        buf = g % 2
        if g + 1 < NGS:
            nbuf = (g + 1) % 2
            for d in sdescs[nbuf]:
                d.wait()
            sdescs[nbuf] = []
            ld[nbuf] = pltpu.async_copy(m_hbm.at[wid, g + 1], mbuf.at[nbuf],
                                        lsem)
        ld[buf].wait()
        sdescs[buf] = [
            pltpu.async_copy(
                mbuf.at[buf, k], acc_sh.at[idx_v.at[g * GCS + k]], ssem,
                add=True,
            )
            for k in range(GCS)
        ]
    for descs in sdescs:
        for d in descs:
            d.wait()
    plsc.subcore_barrier()
    pltpu.sync_copy(acc_sh.at[pl.ds(s * NPT, NPT)], node_v)
    pltpu.sync_copy(node_v, agg_hbm.at[c, pl.ds(s * NPT, NPT)])

    wdesc = [None, None]
    for g in range(NGG):
        buf = g % 2
        if wdesc[buf] is not None:
            wdesc[buf].wait()
        gds = [
            pltpu.async_copy(
                acc_sh.at[idxg_v.at[g * GCG + k]], rows_v.at[buf, k], gsem
            )
            for k in range(GCG)
        ]
        for d in gds:
            d.wait()
        wdesc[buf] = pltpu.async_copy(rows_v.at[buf], asrc_hbm.at[c, s, g],
                                      wsem)
    for d in wdesc:
        if d is not None:
            d.wait()


@functools.partial(
    pl.kernel,
    out_type=jax.ShapeDtypeStruct((NC, N, H), jnp.float32),
    mesh=_mesh,
    compiler_params=pltpu.CompilerParams(use_tc_tiling_on_sc=False),
    scratch_types=[
        pltpu.VMEM((NCH, CHUNK), jnp.int32),
        pltpu.VMEM((2, GC1, CHUNK, H), jnp.float32),
        pltpu.VMEM((NPT, H), jnp.float32),
        pltpu.VMEM_SHARED((N, H), jnp.float32),
        pltpu.SemaphoreType.DMA,
        pltpu.SemaphoreType.DMA,
    ],
)
def _sc_scatter(m_hbm, dst_hbm, zeros_hbm, agg_hbm, idx_v, mbuf, node_v,
                acc_sh, lsem, ssem):
    c = lax.axis_index("c")
    s = lax.axis_index("s")
    wid = s * NC + c
    pltpu.sync_copy(dst_hbm.at[wid], idx_v)
    pltpu.sync_copy(zeros_hbm, node_v)
    pltpu.sync_copy(node_v, acc_sh.at[pl.ds(s * NPT, NPT)])
    plsc.subcore_barrier()

    ld = [None, None]
    sdescs = [[], []]
    ld[0] = pltpu.async_copy(m_hbm.at[wid, 0], mbuf.at[0], lsem)
    for g in range(NG1):
        buf = g % 2
        if g + 1 < NG1:
            nbuf = (g + 1) % 2
            for d in sdescs[nbuf]:
                d.wait()
            sdescs[nbuf] = []
            ld[nbuf] = pltpu.async_copy(m_hbm.at[wid, g + 1], mbuf.at[nbuf],
                                        lsem)
        ld[buf].wait()
        sdescs[buf] = [
            pltpu.async_copy(
                mbuf.at[buf, k], acc_sh.at[idx_v.at[g * GC1 + k]], ssem,
                add=True,
            )
            for k in range(GC1)
        ]
    for descs in sdescs:
        for d in descs:
            d.wait()
    plsc.subcore_barrier()
    pltpu.sync_copy(acc_sh.at[pl.ds(s * NPT, NPT)], node_v)
    pltpu.sync_copy(node_v, agg_hbm.at[c, pl.ds(s * NPT, NPT)])


BM = 6400
BMP = BM // PK
L = PK * H
LW = PK * H * H


def _dot(a, b):
    return jax.lax.dot_general(a, b, (((1,), (0,)), ((), ())),
                               preferred_element_type=jnp.float32)


def _sigmoid(v):
    return 0.5 * jnp.tanh(0.5 * v) + 0.5


def _gru_math(agg, h, wz, wr, wh, uz, ur, uh, bz, br, bh):
    z = _sigmoid(_dot(agg, wz) + _dot(h, uz) + bz)
    r = _sigmoid(_dot(agg, wr) + _dot(h, ur) + br)
    hh = jnp.tanh(_dot(agg, wh) + _dot(r * h, uh) + bh)
    return z * h + (1.0 - z) * hh


def _msg_math(hs, ea, w1c, t16, gsum, w2, b2):
    p2 = _dot(hs, w1c)
    et = _dot(ea, t16)
    return _dot(p2 * et, gsum) + _dot(ea, w2) + b2


def _msg0_body(hs_ref, ea_ref, w1c_ref, t16_ref, g_ref, w2_ref, b2_ref,
               m_ref):
    m_ref[...] = _msg_math(hs_ref[...], ea_ref[...], w1c_ref[...],
                           t16_ref[...], g_ref[...], w2_ref[...], b2_ref[...])


def _msgg_body(hs_ref, a0_ref, a1_ref, ea_ref, w1c_ref, t16_ref, g_ref,
               w2_ref, b2_ref, wz_ref, wr_ref, wh_ref, uz_ref, ur_ref, uh_ref,
               bz_ref, br_ref, bh_ref, m_ref, hsn_ref):
    agg = a0_ref[...] + a1_ref[...]
    hsn = _gru_math(agg, hs_ref[...], wz_ref[...], wr_ref[...], wh_ref[...],
                    uz_ref[...], ur_ref[...], uh_ref[...], bz_ref[...],
                    br_ref[...], bh_ref[...])
    hsn_ref[...] = hsn
    m_ref[...] = _msg_math(hsn, ea_ref[...], w1c_ref[...], t16_ref[...],
                           g_ref[...], w2_ref[...], b2_ref[...])


def _tc_msg0(hs, ea, mw):
    grid = (E // BM,)
    blk = pl.BlockSpec((BMP, L), lambda i: (i, 0))
    zero = lambda i: (0, 0)
    return pl.pallas_call(
        _msg0_body,
        grid=grid,
        in_specs=[
            blk,
            blk,
            pl.BlockSpec((L, LW), zero),
            pl.BlockSpec((L, LW), zero),
            pl.BlockSpec((LW, L), zero),
            pl.BlockSpec((L, L), zero),
            pl.BlockSpec((1, L), zero),
        ],
        out_specs=blk,
        out_shape=jax.ShapeDtypeStruct((EP, L), jnp.float32),
    )(hs, ea, *mw)


def _tc_msg_gru(hs, a0, a1, ea, mw, gw):
    grid = (E // BM,)
    blk = pl.BlockSpec((BMP, L), lambda i: (i, 0))
    zero = lambda i: (0, 0)
    wide_in = pl.BlockSpec((L, LW), zero)
    wide_out = pl.BlockSpec((LW, L), zero)
    sq = pl.BlockSpec((L, L), zero)
    bias = pl.BlockSpec((1, L), zero)
    return pl.pallas_call(
        _msgg_body,
        grid=grid,
        in_specs=[blk, blk, blk, blk,
                  wide_in, wide_in, wide_out, sq, bias,
                  sq, sq, sq, sq, sq, sq, bias, bias, bias],
        out_specs=(blk, blk),
        out_shape=(
            jax.ShapeDtypeStruct((EP, L), jnp.float32),
            jax.ShapeDtypeStruct((EP, L), jnp.float32),
        ),
    )(hs, a0, a1, ea, *mw, *gw)


def _ngru_body(h_ref, a0_ref, a1_ref, wz_ref, wr_ref, wh_ref, uz_ref, ur_ref,
               uh_ref, bz_ref, br_ref, bh_ref, out_ref):
    agg = a0_ref[...] + a1_ref[...]
    out_ref[...] = _gru_math(agg, h_ref[...], wz_ref[...], wr_ref[...],
                             wh_ref[...], uz_ref[...], ur_ref[...],
                             uh_ref[...], bz_ref[...], br_ref[...],
                             bh_ref[...])


def _tc_ngru(h, a0, a1, gw):
    return pl.pallas_call(
        _ngru_body,
        out_shape=jax.ShapeDtypeStruct((NP, L), jnp.float32),
    )(h, a0, a1, *gw)


def kernel(x, edge_index, edge_attr, W1, b1, W2, b2, W_gru, U_gru, b_gru):
    src = edge_index[0].reshape(NW, NCH, CHUNK)
    srcg = edge_index[0].reshape(NS, NCHG, CHUNK)
    dst = edge_index[1].reshape(NW, NCH, CHUNK)
    eap = edge_attr.reshape(EP, L)
    xp = x.reshape(NP, L)

    i8 = jnp.eye(PK, dtype=jnp.float32)
    k8 = lambda w: jnp.kron(i8, w)
    w1c = W1.reshape(DE, H, H).transpose(2, 1, 0).reshape(H, H * H)
    t16 = jnp.tile(jnp.eye(DE, dtype=jnp.float32), (1, H))
    gsum = jnp.repeat(jnp.eye(H, dtype=jnp.float32), H, axis=0)
    mw = (k8(w1c), k8(t16), k8(gsum), k8(W2),
          jnp.tile(b2.reshape(1, H), (1, PK)))

    tile8 = lambda b: jnp.tile(b.reshape(1, H), (1, PK))
    gw = (k8(W_gru[:, :H]), k8(W_gru[:, H:2 * H]), k8(W_gru[:, 2 * H:]),
          k8(U_gru[:, :H]), k8(U_gru[:, H:2 * H]), k8(U_gru[:, 2 * H:]),
          tile8(b_gru[:H]), tile8(b_gru[H:2 * H]), tile8(b_gru[2 * H:]))

    zeros_tile = jnp.zeros((NPT, H), dtype=jnp.float32)

    hs = _sc_gather(x, src).reshape(EP, L)
    hp = xp
    for t in range(TSTEPS):
        if t == 0:
            m = _tc_msg0(hs, eap, mw)
        else:
            m, hs = _tc_msg_gru(hs, a0, a1, eap, mw, gw)
        if t + 1 < TSTEPS:
            agg, asrc = _sc_scatter_gather(
                m.reshape(NW, NGS, GCS, CHUNK, H), dst, srcg, zeros_tile
            )
            a0 = asrc[0].reshape(EP, L)
            a1 = asrc[1].reshape(EP, L)
        else:
            agg = _sc_scatter(
                m.reshape(NW, NG1, GC1, CHUNK, H), dst, zeros_tile
            )
        hp = _tc_ngru(hp, agg[0].reshape(NP, L), agg[1].reshape(NP, L), gw)
    return hp.reshape(N, H)


# speedup vs baseline: 2.5054x; 2.5054x over previous
import functools

import jax
import jax.numpy as jnp
from jax import lax
from jax.experimental import pallas as pl
from jax.experimental.pallas import tpu as pltpu
from jax.experimental.pallas import tpu_sc as plsc

N = 10000
E = 320000
H = 16
DE = 16
TSTEPS = 4
PK = 8
EP = E // PK
NP = N // PK

NC = 2
NS = 16
NW = NC * NS
EW = E // NW
CHUNK = 100
NCH = EW // CHUNK
GC1 = 25
NG1 = NCH // GC1
GC2 = 10
NG2 = NCH // GC2
NPT = N // NS

_mesh = plsc.VectorSubcoreMesh(
    core_axis_name="c", subcore_axis_name="s", num_cores=NC, num_subcores=NS
)


@functools.partial(
    pl.kernel,
    out_type=jax.ShapeDtypeStruct((NW, NG1, GC1, CHUNK, H), jnp.float32),
    mesh=_mesh,
    compiler_params=pltpu.CompilerParams(use_tc_tiling_on_sc=False),
    scratch_types=[
        pltpu.VMEM((NCH, CHUNK), jnp.int32),
        pltpu.VMEM((2, GC1, CHUNK, H), jnp.float32),
        pltpu.SemaphoreType.DMA,
        pltpu.SemaphoreType.DMA,
    ],
)
def _sc_gather(h_hbm, src_hbm, hs_hbm, idx_v, rows_v, gsem, wsem):
    c = lax.axis_index("c")
    s = lax.axis_index("s")
    wid = s * NC + c
    pltpu.sync_copy(src_hbm.at[wid], idx_v)

    wdesc = [None, None]
    for g in range(NG1):
        buf = g % 2
        if wdesc[buf] is not None:
            wdesc[buf].wait()
        gds = [
            pltpu.async_copy(
                h_hbm.at[idx_v.at[g * GC1 + k]], rows_v.at[buf, k], gsem
            )
            for k in range(GC1)
        ]
        for d in gds:
            d.wait()
        wdesc[buf] = pltpu.async_copy(rows_v.at[buf], hs_hbm.at[wid, g], wsem)
    for d in wdesc:
        if d is not None:
            d.wait()


@functools.partial(
    pl.kernel,
    out_type=(
        jax.ShapeDtypeStruct((NW, NG2, GC2, CHUNK, H), jnp.float32),
        jax.ShapeDtypeStruct((NW, NG2, GC2, CHUNK, H), jnp.float32),
    ),
    mesh=_mesh,
    compiler_params=pltpu.CompilerParams(use_tc_tiling_on_sc=False),
    scratch_types=[
        pltpu.VMEM((NCH, CHUNK), jnp.int32),
        pltpu.VMEM((2, GC2, CHUNK, H), jnp.float32),
        pltpu.VMEM((2, GC2, CHUNK, H), jnp.float32),
        pltpu.SemaphoreType.DMA,
        pltpu.SemaphoreType.DMA,
    ],
)
def _sc_gather2(agg_hbm, src_hbm, a0_hbm, a1_hbm, idx_v, r0_v, r1_v, gsem,
                wsem):
    c = lax.axis_index("c")
    s = lax.axis_index("s")
    wid = s * NC + c
    pltpu.sync_copy(src_hbm.at[wid], idx_v)

    wdescs = [[], []]
    for g in range(NG2):
        buf = g % 2
        for d in wdescs[buf]:
            d.wait()
        wdescs[buf] = []
        gds = []
        for k in range(GC2):
            row = idx_v.at[g * GC2 + k]
            gds.append(
                pltpu.async_copy(agg_hbm.at[0].at[row], r0_v.at[buf, k], gsem)
            )
            gds.append(
                pltpu.async_copy(agg_hbm.at[1].at[row], r1_v.at[buf, k], gsem)
            )
        for d in gds:
            d.wait()
        wdescs[buf] = [
            pltpu.async_copy(r0_v.at[buf], a0_hbm.at[wid, g], wsem),
            pltpu.async_copy(r1_v.at[buf], a1_hbm.at[wid, g], wsem),
        ]
    for descs in wdescs:
        for d in descs:
            d.wait()


@functools.partial(
    pl.kernel,
    out_type=jax.ShapeDtypeStruct((NC, N, H), jnp.float32),
    mesh=_mesh,
    compiler_params=pltpu.CompilerParams(use_tc_tiling_on_sc=False),
    scratch_types=[
        pltpu.VMEM((NCH, CHUNK), jnp.int32),
        pltpu.VMEM((2, GC1, CHUNK, H), jnp.float32),
        pltpu.VMEM((NPT, H), jnp.float32),
        pltpu.VMEM_SHARED((N, H), jnp.float32),
        pltpu.SemaphoreType.DMA,
        pltpu.SemaphoreType.DMA,
    ],
)
def _sc_scatter(m_hbm, dst_hbm, zeros_hbm, agg_hbm, idx_v, mbuf, node_v,
                acc_sh, lsem, ssem):
    c = lax.axis_index("c")
    s = lax.axis_index("s")
    wid = s * NC + c
    pltpu.sync_copy(dst_hbm.at[wid], idx_v)
    pltpu.sync_copy(zeros_hbm, node_v)
    pltpu.sync_copy(node_v, acc_sh.at[pl.ds(s * NPT, NPT)])
    plsc.subcore_barrier()

    ld = [None, None]
    sdescs = [[], []]
    ld[0] = pltpu.async_copy(m_hbm.at[wid, 0], mbuf.at[0], lsem)
    for g in range(NG1):
        buf = g % 2
        if g + 1 < NG1:
            nbuf = (g + 1) % 2
            for d in sdescs[nbuf]:
                d.wait()
            sdescs[nbuf] = []
            ld[nbuf] = pltpu.async_copy(m_hbm.at[wid, g + 1], mbuf.at[nbuf],
                                        lsem)
        ld[buf].wait()
        sdescs[buf] = [
            pltpu.async_copy(
                mbuf.at[buf, k], acc_sh.at[idx_v.at[g * GC1 + k]], ssem,
                add=True,
            )
            for k in range(GC1)
        ]
    for descs in sdescs:
        for d in descs:
            d.wait()
    plsc.subcore_barrier()
    pltpu.sync_copy(acc_sh.at[pl.ds(s * NPT, NPT)], node_v)
    pltpu.sync_copy(node_v, agg_hbm.at[c, pl.ds(s * NPT, NPT)])


BM = 12800
BMP = BM // PK
L = PK * H
LW = PK * H * H


def _dot(a, b):
    return jax.lax.dot_general(a, b, (((1,), (0,)), ((), ())),
                               preferred_element_type=jnp.float32)


def _sigmoid(v):
    return 0.5 * jnp.tanh(0.5 * v) + 0.5


def _gru_math(agg, h, wz, wr, wh, uz, ur, uh, bz, br, bh):
    z = _sigmoid(_dot(agg, wz) + _dot(h, uz) + bz)
    r = _sigmoid(_dot(agg, wr) + _dot(h, ur) + br)
    hh = jnp.tanh(_dot(agg, wh) + _dot(r * h, uh) + bh)
    return z * h + (1.0 - z) * hh


def _msg_math(hs, ea, w1c, t16, gsum, w2, b2):
    p2 = _dot(hs, w1c)
    et = _dot(ea, t16)
    return _dot(p2 * et, gsum) + _dot(ea, w2) + b2


def _msg0_body(hs_ref, ea_ref, w1c_ref, t16_ref, g_ref, w2_ref, b2_ref,
               m_ref):
    m_ref[...] = _msg_math(hs_ref[...], ea_ref[...], w1c_ref[...],
                           t16_ref[...], g_ref[...], w2_ref[...], b2_ref[...])


def _msgg_body(hs_ref, a0_ref, a1_ref, ea_ref, w1c_ref, t16_ref, g_ref,
               w2_ref, b2_ref, wz_ref, wr_ref, wh_ref, uz_ref, ur_ref, uh_ref,
               bz_ref, br_ref, bh_ref, m_ref, hsn_ref):
    agg = a0_ref[...] + a1_ref[...]
    hsn = _gru_math(agg, hs_ref[...], wz_ref[...], wr_ref[...], wh_ref[...],
                    uz_ref[...], ur_ref[...], uh_ref[...], bz_ref[...],
                    br_ref[...], bh_ref[...])
    hsn_ref[...] = hsn
    m_ref[...] = _msg_math(hsn, ea_ref[...], w1c_ref[...], t16_ref[...],
                           g_ref[...], w2_ref[...], b2_ref[...])


def _tc_msg0(hs, ea, mw):
    grid = (E // BM,)
    blk = pl.BlockSpec((BMP, L), lambda i: (i, 0))
    zero = lambda i: (0, 0)
    return pl.pallas_call(
        _msg0_body,
        grid=grid,
        in_specs=[
            blk,
            blk,
            pl.BlockSpec((L, LW), zero),
            pl.BlockSpec((L, LW), zero),
            pl.BlockSpec((LW, L), zero),
            pl.BlockSpec((L, L), zero),
            pl.BlockSpec((1, L), zero),
        ],
        out_specs=blk,
        out_shape=jax.ShapeDtypeStruct((EP, L), jnp.float32),
    )(hs, ea, *mw)


def _tc_msg_gru(hs, a0, a1, ea, mw, gw):
    grid = (E // BM,)
    blk = pl.BlockSpec((BMP, L), lambda i: (i, 0))
    zero = lambda i: (0, 0)
    wide_in = pl.BlockSpec((L, LW), zero)
    wide_out = pl.BlockSpec((LW, L), zero)
    sq = pl.BlockSpec((L, L), zero)
    bias = pl.BlockSpec((1, L), zero)
    return pl.pallas_call(
        _msgg_body,
        grid=grid,
        in_specs=[blk, blk, blk, blk,
                  wide_in, wide_in, wide_out, sq, bias,
                  sq, sq, sq, sq, sq, sq, bias, bias, bias],
        out_specs=(blk, blk),
        out_shape=(
            jax.ShapeDtypeStruct((EP, L), jnp.float32),
            jax.ShapeDtypeStruct((EP, L), jnp.float32),
        ),
    )(hs, a0, a1, ea, *mw, *gw)


def _ngru_body(h_ref, a0_ref, a1_ref, wz_ref, wr_ref, wh_ref, uz_ref, ur_ref,
               uh_ref, bz_ref, br_ref, bh_ref, out_ref):
    agg = a0_ref[...] + a1_ref[...]
    out_ref[...] = _gru_math(agg, h_ref[...], wz_ref[...], wr_ref[...],
                             wh_ref[...], uz_ref[...], ur_ref[...],
                             uh_ref[...], bz_ref[...], br_ref[...],
                             bh_ref[...])


def _tc_ngru(h, a0, a1, gw):
    return pl.pallas_call(
        _ngru_body,
        out_shape=jax.ShapeDtypeStruct((NP, L), jnp.float32),
    )(h, a0, a1, *gw)


def kernel(x, edge_index, edge_attr, W1, b1, W2, b2, W_gru, U_gru, b_gru):
    src = edge_index[0].reshape(NW, NCH, CHUNK)
    dst = edge_index[1].reshape(NW, NCH, CHUNK)
    eap = edge_attr.reshape(EP, L)
    xp = x.reshape(NP, L)

    i8 = jnp.eye(PK, dtype=jnp.float32)
    k8 = lambda w: jnp.kron(i8, w)
    w1c = W1.reshape(DE, H, H).transpose(2, 1, 0).reshape(H, H * H)
    t16 = jnp.tile(jnp.eye(DE, dtype=jnp.float32), (1, H))
    gsum = jnp.repeat(jnp.eye(H, dtype=jnp.float32), H, axis=0)
    mw = (k8(w1c), k8(t16), k8(gsum), k8(W2),
          jnp.tile(b2.reshape(1, H), (1, PK)))

    tile8 = lambda b: jnp.tile(b.reshape(1, H), (1, PK))
    gw = (k8(W_gru[:, :H]), k8(W_gru[:, H:2 * H]), k8(W_gru[:, 2 * H:]),
          k8(U_gru[:, :H]), k8(U_gru[:, H:2 * H]), k8(U_gru[:, 2 * H:]),
          tile8(b_gru[:H]), tile8(b_gru[H:2 * H]), tile8(b_gru[2 * H:]))

    zeros_tile = jnp.zeros((NPT, H), dtype=jnp.float32)

    hs = _sc_gather(x, src).reshape(EP, L)
    hp = xp
    for t in range(TSTEPS):
        if t == 0:
            m = _tc_msg0(hs, eap, mw)
        else:
            m, hs = _tc_msg_gru(hs, a0, a1, eap, mw, gw)
        agg = _sc_scatter(m.reshape(NW, NG1, GC1, CHUNK, H), dst, zeros_tile)
        if t + 1 < TSTEPS:
            a0r, a1r = _sc_gather2(agg, src)
            a0 = a0r.reshape(EP, L)
            a1 = a1r.reshape(EP, L)
        hp = _tc_ngru(hp, agg[0].reshape(NP, L), agg[1].reshape(NP, L), gw)
    return hp.reshape(N, H)
